# baseline (speedup 1.0000x reference)
import functools

import jax
import jax.numpy as jnp
from jax import lax
from jax.experimental import pallas as pl
from jax.experimental.pallas import tpu as pltpu
from jax.experimental.pallas import tpu_sc as plsc

N = 10000
E = 160000
D_IN = 256
D_HID = 256
D_OUT = 16
NB = 4
NS = 4
NT = 16
NC = 2
LANES = 16

TN = 2000
GRID = N // TN

NPAD = 10240
RPT = NPAD // NT
NQ = 2560
QRPT = NQ // NT

EPC = E // NC
EPT = EPC // NT
EPTP = 5008
LL1 = 128
LROWS = NQ // LL1
ZB1 = 32

EPT2 = EPC // NT
LL2 = 160
NSUP2 = 32
SLOTS2 = NSUP2 * LL2


def _l1_body(x_ref, w_ref, comp_ref, out_ref):
    x = x_ref[...]
    w = w_ref[...]
    comp = comp_ref[...]
    xb = [jnp.dot(x, w[b * D_IN:(b + 1) * D_IN, :],
                  preferred_element_type=jnp.float32) for b in range(NB)]
    for r in range(NS):
        acc = comp[r:r + 1, 0:1] * xb[0]
        for b in range(1, NB):
            acc = acc + comp[r:r + 1, b:b + 1] * xb[b]
        out_ref[r] = acc


def _l1_matmul(x, W1, W1_comp):
    return pl.pallas_call(
        _l1_body,
        grid=(GRID,),
        in_specs=[
            pl.BlockSpec((TN, D_IN), lambda i: (i, 0)),
            pl.BlockSpec((NB * D_IN, D_HID), lambda i: (0, 0)),
            pl.BlockSpec((NS, NB), lambda i: (0, 0)),
        ],
        out_specs=pl.BlockSpec((NS, TN, D_HID), lambda i: (0, i, 0)),
        out_shape=jax.ShapeDtypeStruct((NS, N, D_HID), jnp.float32),
    )(x, W1, W1_comp)


def _l2_body(p_ref, w_ref, comp_ref, out_ref):
    h = jnp.maximum(p_ref[0, 0] + p_ref[1, 0], 0.0)
    w = w_ref[...]
    comp = comp_ref[...]
    wcat = jnp.concatenate(
        [w[b * D_HID:(b + 1) * D_HID, :] for b in range(NB)], axis=1)
    hb = jnp.dot(h, wcat, preferred_element_type=jnp.float32)
    for r in range(NS):
        acc = comp[r:r + 1, 0:1] * hb[:, 0:D_OUT]
        for b in range(1, NB):
            acc = acc + comp[r:r + 1, b:b + 1] * hb[:, b * D_OUT:(b + 1) * D_OUT]
        out_ref[r] = acc


def _l2_matmul(parts, W2, W2_comp):
    tn2 = 1280
    return pl.pallas_call(
        _l2_body,
        grid=(NPAD // tn2,),
        in_specs=[
            pl.BlockSpec((NC, 1, tn2, D_HID), lambda i: (0, i // 2, i % 2, 0)),
            pl.BlockSpec((NB * D_HID, D_OUT), lambda i: (0, 0)),
            pl.BlockSpec((NS, NB), lambda i: (0, 0)),
        ],
        out_specs=pl.BlockSpec((NS, tn2, D_OUT), lambda i: (0, i, 0)),
        out_shape=jax.ShapeDtypeStruct((NS, NPAD, D_OUT), jnp.float32),
    )(parts, W2, W2_comp)


def _softmax_body(p_ref, out_ref):
    s = p_ref[0] + p_ref[1]
    m = jnp.max(s, axis=1, keepdims=True)
    e = jnp.exp(s - m)
    out_ref[...] = e / jnp.sum(e, axis=1, keepdims=True)


def _softmax_sum(parts):
    return pl.pallas_call(
        _softmax_body,
        grid=(GRID,),
        in_specs=[pl.BlockSpec((NC, TN, D_OUT), lambda i: (0, i, 0))],
        out_specs=pl.BlockSpec((TN, D_OUT), lambda i: (i, 0)),
        out_shape=jax.ShapeDtypeStruct((N, D_OUT), jnp.float32),
    )(parts)


_MESH = plsc.VectorSubcoreMesh(core_axis_name="c", subcore_axis_name="s")


@functools.partial(
    pl.kernel,
    out_type=jax.ShapeDtypeStruct((NC, 4, NQ, D_HID), jnp.float32),
    mesh=_MESH,
    compiler_params=pltpu.CompilerParams(
        use_tc_tiling_on_sc=False, needs_layout_passes=False),
    scratch_types=[
        pltpu.VMEM((EPTP,), jnp.int32),
        pltpu.VMEM((EPTP,), jnp.int32),
        pltpu.VMEM((EPTP,), jnp.int32),
        pltpu.VMEM((4 * LROWS, LL1), jnp.int32),
        pltpu.VMEM((4 * LROWS, LL1), jnp.int32),
        pltpu.VMEM((LL1, D_HID), jnp.float32),
        pltpu.VMEM((ZB1, D_HID), jnp.float32),
        pltpu.VMEM((4, LANES), jnp.int32),
        pltpu.VMEM_SHARED((NQ + NT, D_HID), jnp.float32),
        pltpu.SemaphoreType.DMA,
    ],
)
def _agg1(y1_hbm, src_hbm, dst_hbm, type_hbm, out_hbm,
          srcv, dstv, typev, glist, dlist, rows, zbuf, cnt, acc, sem):
    c = lax.axis_index("c")
    s = lax.axis_index("s")
    ebase = c * EPC + s * EPT

    pltpu.sync_copy(src_hbm.at[pl.ds(ebase, EPT)], srcv.at[pl.ds(0, EPT)])
    pltpu.sync_copy(dst_hbm.at[pl.ds(ebase, EPT)], dstv.at[pl.ds(0, EPT)])
    pltpu.sync_copy(type_hbm.at[pl.ds(ebase, EPT)], typev.at[pl.ds(0, EPT)])

    lane = lax.broadcasted_iota(jnp.int32, (LANES,), 0)
    zv = jnp.zeros((LANES,), jnp.float32)
    zi = jnp.zeros((LANES,), jnp.int32)

    for q in range(4):
        cnt[q] = zi

    def scan(i, carry):
        off = i * LANES
        sv = srcv[pl.ds(off, LANES)]
        tv = typev[pl.ds(off, LANES)]
        dv = dstv[pl.ds(off, LANES)]
        valid = (off + lane) < EPT
        g = tv * N + sv
        one = jnp.full((LANES,), 1, jnp.int32)
        qv = (jnp.where(dv >= NQ, one, 0) + jnp.where(dv >= 2 * NQ, one, 0)
              + jnp.where(dv >= 3 * NQ, one, 0))
        for q in range(4):
            cq = cnt[q]
            m = (qv == q) & valid
            mi = jnp.where(m, one, 0)
            pos = jnp.minimum(cq + plsc.cumsum(mi) - 1, NQ - 1)
            lrow = q * LROWS + pos // LL1
            lcol = pos % LL1
            plsc.store_scatter(glist, [lrow, lcol], g, mask=m)
            plsc.store_scatter(dlist, [lrow, lcol], dv - q * NQ, mask=m)
            cnt[q] = cq + plsc.all_reduce_population_count(m)
        return carry

    lax.fori_loop(0, EPTP // LANES, scan, 0)

    def pad(i, carry):
        for q in range(4):
            cq_vec = cnt[q]
            pos = cq_vec + i * LANES + lane
            m = pos < NQ
            pos = jnp.minimum(pos, NQ - 1)
            lrow = q * LROWS + pos // LL1
            lcol = pos % LL1
            plsc.store_scatter(glist, [lrow, lcol], zi, mask=m)
            plsc.store_scatter(dlist, [lrow, lcol], zi + NQ + s, mask=m)
        return carry

    lax.fori_loop(0, NQ // LANES, pad, 0)

    def zfill(i, carry):
        for k in range(D_HID // LANES):
            zbuf[i, pl.ds(k * LANES, LANES)] = zv
        return carry

    r0 = s * QRPT
    for q in range(4):
        lax.fori_loop(0, ZB1, zfill, 0)
        for z in range(QRPT // ZB1):
            pltpu.sync_copy(zbuf, acc.at[pl.ds(r0 + z * ZB1, ZB1)])

        plsc.subcore_barrier()

        def step(m, carry):
            pltpu.async_copy(
                y1_hbm.at[glist.at[q * LROWS + m]], rows, sem).wait()
            pltpu.sync_copy(rows, acc.at[dlist.at[q * LROWS + m]], add=True)
            return carry

        lax.fori_loop(0, LROWS, step, 0)

        plsc.subcore_barrier()

        for z in range(QRPT // ZB1):
            pltpu.sync_copy(acc.at[pl.ds(r0 + z * ZB1, ZB1)], zbuf)
            pltpu.sync_copy(zbuf, out_hbm.at[c, q, pl.ds(r0 + z * ZB1, ZB1)])


@functools.partial(
    pl.kernel,
    out_type=jax.ShapeDtypeStruct((NC, NPAD, D_OUT), jnp.float32),
    mesh=_MESH,
    compiler_params=pltpu.CompilerParams(use_tc_tiling_on_sc=False),
    scratch_types=[
        pltpu.VMEM((SLOTS2,), jnp.int32),
        pltpu.VMEM((SLOTS2,), jnp.int32),
        pltpu.VMEM((SLOTS2,), jnp.int32),
        pltpu.VMEM((NSUP2, LL2), jnp.int32),
        pltpu.VMEM((NSUP2, LL2), jnp.int32),
        pltpu.VMEM((LL2, D_OUT), jnp.float32),
        pltpu.VMEM((RPT, D_OUT), jnp.float32),
        pltpu.VMEM_SHARED((NPAD, D_OUT), jnp.float32),
        pltpu.SemaphoreType.DMA,
    ],
)
def _agg2(y2_hbm, src_hbm, dst_hbm, type_hbm, out_hbm,
          srcv, dstv, typev, gidx, didx, rows, zbuf, acc, sem):
    c = lax.axis_index("c")
    s = lax.axis_index("s")
    ebase = c * EPC + s * EPT2

    pltpu.sync_copy(src_hbm.at[pl.ds(ebase, EPT2)], srcv.at[pl.ds(0, EPT2)])
    pltpu.sync_copy(dst_hbm.at[pl.ds(ebase, EPT2)], dstv.at[pl.ds(0, EPT2)])
    pltpu.sync_copy(type_hbm.at[pl.ds(ebase, EPT2)], typev.at[pl.ds(0, EPT2)])

    lane = lax.broadcasted_iota(jnp.int32, (LANES,), 0)
    zv = jnp.zeros((LANES,), jnp.float32)

    def fill(j, carry):
        for k in range(LL2 // LANES):
            off = j * LL2 + k * LANES
            sv = srcv[pl.ds(off, LANES)]
            tv = typev[pl.ds(off, LANES)]
            dv = dstv[pl.ds(off, LANES)]
            valid = (off + lane) < EPT2
            gidx[j, pl.ds(k * LANES, LANES)] = jnp.where(valid, tv * NPAD + sv, 0)
            didx[j, pl.ds(k * LANES, LANES)] = jnp.where(valid, dv, N + s)
        return carry

    lax.fori_loop(0, NSUP2, fill, 0)

    def zfill(i, carry):
        zbuf[i] = zv
        return carry

    lax.fori_loop(0, RPT, zfill, 0)
    pltpu.sync_copy(zbuf, acc.at[pl.ds(s * RPT, RPT)])

    plsc.subcore_barrier()

    def step(m, carry):
        pltpu.async_copy(y2_hbm.at[gidx.at[m]], rows, sem).wait()
        pltpu.sync_copy(rows, acc.at[didx.at[m]], add=True)
        return carry

    lax.fori_loop(0, NSUP2, step, 0)

    plsc.subcore_barrier()

    pltpu.sync_copy(acc.at[pl.ds(s * RPT, RPT)], zbuf)
    pltpu.sync_copy(zbuf, out_hbm.at[c, pl.ds(s * RPT, RPT)])


def kernel(x, edge_index, edge_type, W1, W1_comp, W2, W2_comp):
    src = edge_index[0]
    dst = edge_index[1]
    y1 = _l1_matmul(x, W1, W1_comp)
    y1s = y1.reshape(NS * N, D_HID)
    h1p = _agg1(y1s, src, dst, edge_type)
    y2 = _l2_matmul(h1p, W2, W2_comp)
    y2s = y2.reshape(NS * NPAD, D_OUT)
    parts = _agg2(y2s, src, dst, edge_type)
    return _softmax_sum(parts)


# speedup vs baseline: 9.9732x; 9.9732x over previous
import functools

import jax
import jax.numpy as jnp
from jax import lax
from jax.experimental import pallas as pl
from jax.experimental.pallas import tpu as pltpu
from jax.experimental.pallas import tpu_sc as plsc

N = 10000
E = 160000
D_IN = 256
D_HID = 256
D_OUT = 16
NB = 4
NS = 4
NT = 16
NC = 2
LANES = 16

TN = 2000
GRID = N // TN

NPAD = 10240
RPT = NPAD // NT
NQ = 2560
QRPT = NQ // NT

EPC = E // NC
EPT = EPC // NT
EPTP = 5008
LL1 = 128
LROWS = NQ // LL1
ZB1 = 32

EPT2 = EPC // NT
LL2 = 160
NSUP2 = 32
SLOTS2 = NSUP2 * LL2


def _l1_body(x_ref, w_ref, comp_ref, out_ref):
    x = x_ref[...]
    w = w_ref[...]
    comp = comp_ref[...]
    xb = [jnp.dot(x, w[b * D_IN:(b + 1) * D_IN, :],
                  preferred_element_type=jnp.float32) for b in range(NB)]
    for r in range(NS):
        acc = comp[r:r + 1, 0:1] * xb[0]
        for b in range(1, NB):
            acc = acc + comp[r:r + 1, b:b + 1] * xb[b]
        out_ref[r] = acc


def _l1_matmul(x, W1, W1_comp):
    return pl.pallas_call(
        _l1_body,
        grid=(GRID,),
        in_specs=[
            pl.BlockSpec((TN, D_IN), lambda i: (i, 0)),
            pl.BlockSpec((NB * D_IN, D_HID), lambda i: (0, 0)),
            pl.BlockSpec((NS, NB), lambda i: (0, 0)),
        ],
        out_specs=pl.BlockSpec((NS, TN, D_HID), lambda i: (0, i, 0)),
        out_shape=jax.ShapeDtypeStruct((NS, N, D_HID), jnp.float32),
    )(x, W1, W1_comp)


def _l2_body(p_ref, w_ref, comp_ref, out_ref):
    h = jnp.maximum(p_ref[0, 0] + p_ref[1, 0], 0.0)
    w = w_ref[...]
    comp = comp_ref[...]
    wcat = jnp.concatenate(
        [w[b * D_HID:(b + 1) * D_HID, :] for b in range(NB)], axis=1)
    hb = jnp.dot(h, wcat, preferred_element_type=jnp.float32)
    for r in range(NS):
        acc = comp[r:r + 1, 0:1] * hb[:, 0:D_OUT]
        for b in range(1, NB):
            acc = acc + comp[r:r + 1, b:b + 1] * hb[:, b * D_OUT:(b + 1) * D_OUT]
        out_ref[r] = acc


def _l2_matmul(parts, W2, W2_comp):
    tn2 = 1280
    return pl.pallas_call(
        _l2_body,
        grid=(NPAD // tn2,),
        in_specs=[
            pl.BlockSpec((NC, 1, tn2, D_HID), lambda i: (0, i // 2, i % 2, 0)),
            pl.BlockSpec((NB * D_HID, D_OUT), lambda i: (0, 0)),
            pl.BlockSpec((NS, NB), lambda i: (0, 0)),
        ],
        out_specs=pl.BlockSpec((NS, tn2, D_OUT), lambda i: (0, i, 0)),
        out_shape=jax.ShapeDtypeStruct((NS, NPAD, D_OUT), jnp.float32),
    )(parts, W2, W2_comp)


def _softmax_body(p_ref, out_ref):
    s = p_ref[0] + p_ref[1]
    m = jnp.max(s, axis=1, keepdims=True)
    e = jnp.exp(s - m)
    out_ref[...] = e / jnp.sum(e, axis=1, keepdims=True)


def _softmax_sum(parts):
    return pl.pallas_call(
        _softmax_body,
        grid=(GRID,),
        in_specs=[pl.BlockSpec((NC, TN, D_OUT), lambda i: (0, i, 0))],
        out_specs=pl.BlockSpec((TN, D_OUT), lambda i: (i, 0)),
        out_shape=jax.ShapeDtypeStruct((N, D_OUT), jnp.float32),
    )(parts)


_MESH = plsc.VectorSubcoreMesh(core_axis_name="c", subcore_axis_name="s")


@functools.partial(
    pl.kernel,
    out_type=jax.ShapeDtypeStruct((NC, 4, NQ, D_HID), jnp.float32),
    mesh=_MESH,
    compiler_params=pltpu.CompilerParams(
        use_tc_tiling_on_sc=False, needs_layout_passes=False),
    scratch_types=[
        pltpu.VMEM((EPTP,), jnp.int32),
        pltpu.VMEM((EPTP,), jnp.int32),
        pltpu.VMEM((EPTP,), jnp.int32),
        pltpu.VMEM((4 * LROWS, LL1), jnp.int32),
        pltpu.VMEM((4 * LROWS, LL1), jnp.int32),
        pltpu.VMEM((LL1, D_HID), jnp.float32),
        pltpu.VMEM((ZB1, D_HID), jnp.float32),
        pltpu.VMEM((4, LANES), jnp.int32),
        pltpu.VMEM_SHARED((NQ + NT, D_HID), jnp.float32),
        pltpu.SemaphoreType.DMA,
    ],
)
def _agg1(y1_hbm, src_hbm, dst_hbm, type_hbm, out_hbm,
          srcv, dstv, typev, glist, dlist, rows, zbuf, cnt, acc, sem):
    c = lax.axis_index("c")
    s = lax.axis_index("s")
    ebase = c * EPC + s * EPT

    pltpu.sync_copy(src_hbm.at[pl.ds(ebase, EPT)], srcv.at[pl.ds(0, EPT)])
    pltpu.sync_copy(dst_hbm.at[pl.ds(ebase, EPT)], dstv.at[pl.ds(0, EPT)])
    pltpu.sync_copy(type_hbm.at[pl.ds(ebase, EPT)], typev.at[pl.ds(0, EPT)])

    lane = lax.broadcasted_iota(jnp.int32, (LANES,), 0)
    zv = jnp.zeros((LANES,), jnp.float32)
    zi = jnp.zeros((LANES,), jnp.int32)

    for q in range(4):
        cnt[q] = zi

    def scan(i, carry):
        off = i * LANES
        sv = srcv[pl.ds(off, LANES)]
        tv = typev[pl.ds(off, LANES)]
        dv = dstv[pl.ds(off, LANES)]
        valid = (off + lane) < EPT
        g = tv * N + sv
        one = jnp.full((LANES,), 1, jnp.int32)
        qv = (jnp.where(dv >= NQ, one, 0) + jnp.where(dv >= 2 * NQ, one, 0)
              + jnp.where(dv >= 3 * NQ, one, 0))
        for q in range(4):
            cq = cnt[q]
            m = (qv == q) & valid
            mi = jnp.where(m, one, 0)
            pos = jnp.minimum(cq + plsc.cumsum(mi) - 1, NQ - 1)
            lrow = q * LROWS + pos // LL1
            lcol = pos % LL1
            plsc.store_scatter(glist, [lrow, lcol], g, mask=m)
            plsc.store_scatter(dlist, [lrow, lcol], dv - q * NQ, mask=m)
            cnt[q] = cq + plsc.all_reduce_population_count(m)
        return carry

    lax.fori_loop(0, EPTP // LANES, scan, 0)

    nops = []
    for q in range(4):
        cq_vec = cnt[q]
        cq = jnp.max(cq_vec)
        nq_ops = (cq + LL1 - 1) // LL1
        nops.append(nq_ops)
        target = nq_ops * LL1
        for k in range(LL1 // LANES):
            pos = cq_vec + k * LANES + lane
            m = pos < target
            pos = jnp.minimum(pos, NQ - 1)
            lrow = q * LROWS + pos // LL1
            lcol = pos % LL1
            plsc.store_scatter(glist, [lrow, lcol], zi, mask=m)
            plsc.store_scatter(dlist, [lrow, lcol], zi + NQ + s, mask=m)

    def zfill(i, carry):
        for k in range(D_HID // LANES):
            zbuf[i, pl.ds(k * LANES, LANES)] = zv
        return carry

    r0 = s * QRPT
    for q in range(4):
        lax.fori_loop(0, ZB1, zfill, 0)
        for z in range(QRPT // ZB1):
            pltpu.sync_copy(zbuf, acc.at[pl.ds(r0 + z * ZB1, ZB1)])

        plsc.subcore_barrier()

        nq_ops = nops[q]

        def step(m, carry):
            @pl.when(m < nq_ops)
            def _():
                pltpu.async_copy(
                    y1_hbm.at[glist.at[q * LROWS + m]], rows, sem).wait()
                pltpu.sync_copy(rows, acc.at[dlist.at[q * LROWS + m]], add=True)
            return carry

        lax.fori_loop(0, LROWS, step, 0)

        plsc.subcore_barrier()

        for z in range(QRPT // ZB1):
            pltpu.sync_copy(acc.at[pl.ds(r0 + z * ZB1, ZB1)], zbuf)
            pltpu.sync_copy(zbuf, out_hbm.at[c, q, pl.ds(r0 + z * ZB1, ZB1)])


@functools.partial(
    pl.kernel,
    out_type=jax.ShapeDtypeStruct((NC, NPAD, D_OUT), jnp.float32),
    mesh=_MESH,
    compiler_params=pltpu.CompilerParams(use_tc_tiling_on_sc=False),
    scratch_types=[
        pltpu.VMEM((SLOTS2,), jnp.int32),
        pltpu.VMEM((SLOTS2,), jnp.int32),
        pltpu.VMEM((SLOTS2,), jnp.int32),
        pltpu.VMEM((NSUP2, LL2), jnp.int32),
        pltpu.VMEM((NSUP2, LL2), jnp.int32),
        pltpu.VMEM((LL2, D_OUT), jnp.float32),
        pltpu.VMEM((RPT, D_OUT), jnp.float32),
        pltpu.VMEM_SHARED((NPAD, D_OUT), jnp.float32),
        pltpu.SemaphoreType.DMA,
    ],
)
def _agg2(y2_hbm, src_hbm, dst_hbm, type_hbm, out_hbm,
          srcv, dstv, typev, gidx, didx, rows, zbuf, acc, sem):
    c = lax.axis_index("c")
    s = lax.axis_index("s")
    ebase = c * EPC + s * EPT2

    pltpu.sync_copy(src_hbm.at[pl.ds(ebase, EPT2)], srcv.at[pl.ds(0, EPT2)])
    pltpu.sync_copy(dst_hbm.at[pl.ds(ebase, EPT2)], dstv.at[pl.ds(0, EPT2)])
    pltpu.sync_copy(type_hbm.at[pl.ds(ebase, EPT2)], typev.at[pl.ds(0, EPT2)])

    lane = lax.broadcasted_iota(jnp.int32, (LANES,), 0)
    zv = jnp.zeros((LANES,), jnp.float32)

    def fill(j, carry):
        for k in range(LL2 // LANES):
            off = j * LL2 + k * LANES
            sv = srcv[pl.ds(off, LANES)]
            tv = typev[pl.ds(off, LANES)]
            dv = dstv[pl.ds(off, LANES)]
            valid = (off + lane) < EPT2
            gidx[j, pl.ds(k * LANES, LANES)] = jnp.where(valid, tv * NPAD + sv, 0)
            didx[j, pl.ds(k * LANES, LANES)] = jnp.where(valid, dv, N + s)
        return carry

    lax.fori_loop(0, NSUP2, fill, 0)

    def zfill(i, carry):
        zbuf[i] = zv
        return carry

    lax.fori_loop(0, RPT, zfill, 0)
    pltpu.sync_copy(zbuf, acc.at[pl.ds(s * RPT, RPT)])

    plsc.subcore_barrier()

    def step(m, carry):
        pltpu.async_copy(y2_hbm.at[gidx.at[m]], rows, sem).wait()
        pltpu.sync_copy(rows, acc.at[didx.at[m]], add=True)
        return carry

    lax.fori_loop(0, NSUP2, step, 0)

    plsc.subcore_barrier()

    pltpu.sync_copy(acc.at[pl.ds(s * RPT, RPT)], zbuf)
    pltpu.sync_copy(zbuf, out_hbm.at[c, pl.ds(s * RPT, RPT)])


def kernel(x, edge_index, edge_type, W1, W1_comp, W2, W2_comp):
    src = edge_index[0]
    dst = edge_index[1]
    y1 = _l1_matmul(x, W1, W1_comp)
    y1s = y1.reshape(NS * N, D_HID)
    h1p = _agg1(y1s, src, dst, edge_type)
    y2 = _l2_matmul(h1p, W2, W2_comp)
    y2s = y2.reshape(NS * NPAD, D_OUT)
    parts = _agg2(y2s, src, dst, edge_type)
    return _softmax_sum(parts)


# speedup vs baseline: 13.8824x; 1.3920x over previous
import functools

import jax
import jax.numpy as jnp
from jax import lax
from jax.experimental import pallas as pl
from jax.experimental.pallas import tpu as pltpu
from jax.experimental.pallas import tpu_sc as plsc

N = 10000
E = 160000
D_IN = 256
D_HID = 256
D_OUT = 16
NB = 4
NS = 4
NT = 16
NC = 2
LANES = 16

TN = 2000
GRID = N // TN

NPAD = 10240
RPT = NPAD // NT

EPT1 = E // NT
LL1 = 160
NSUP1 = 63
SLOTS1 = NSUP1 * LL1
ZR1 = 128

EPC = E // NC
EPT2 = EPC // NT
K2 = 80
SB2 = 16
NSUP2 = 4
NCH2 = NSUP2 * SB2
SLOTS2 = NCH2 * K2


def _l1_body(x_ref, w_ref, comp_ref, out_ref):
    x = x_ref[...]
    w = w_ref[...]
    comp = comp_ref[...]
    xb = [jnp.dot(x, w[b * D_IN:(b + 1) * D_IN, :],
                  preferred_element_type=jnp.float32) for b in range(NB)]
    for r in range(NS):
        acc = comp[r:r + 1, 0:1] * xb[0]
        for b in range(1, NB):
            acc = acc + comp[r:r + 1, b:b + 1] * xb[b]
        out_ref[r] = acc


def _l1_matmul(x, W1, W1_comp):
    return pl.pallas_call(
        _l1_body,
        grid=(GRID,),
        in_specs=[
            pl.BlockSpec((TN, D_IN), lambda i: (i, 0)),
            pl.BlockSpec((NB * D_IN, D_HID), lambda i: (0, 0)),
            pl.BlockSpec((NS, NB), lambda i: (0, 0)),
        ],
        out_specs=pl.BlockSpec((NS, TN, D_HID), lambda i: (0, i, 0)),
        out_shape=jax.ShapeDtypeStruct((NS, N, D_HID), jnp.float32),
    )(x, W1, W1_comp)


def _l2_body(p_ref, w_ref, comp_ref, out_ref):
    h = jnp.maximum(
        jnp.concatenate([p_ref[q] for q in range(4)], axis=1), 0.0)
    w = w_ref[...]
    comp = comp_ref[...]
    wcat = jnp.concatenate(
        [w[b * D_HID:(b + 1) * D_HID, :] for b in range(NB)], axis=1)
    hb = jnp.dot(h, wcat, preferred_element_type=jnp.float32)
    for r in range(NS):
        acc = comp[r:r + 1, 0:1] * hb[:, 0:D_OUT]
        for b in range(1, NB):
            acc = acc + comp[r:r + 1, b:b + 1] * hb[:, b * D_OUT:(b + 1) * D_OUT]
        out_ref[r] = acc


def _l2_matmul(parts, W2, W2_comp):
    return pl.pallas_call(
        _l2_body,
        grid=(GRID,),
        in_specs=[
            pl.BlockSpec((4, TN, 64), lambda i: (0, i, 0)),
            pl.BlockSpec((NB * D_HID, D_OUT), lambda i: (0, 0)),
            pl.BlockSpec((NS, NB), lambda i: (0, 0)),
        ],
        out_specs=pl.BlockSpec((NS, TN, D_OUT), lambda i: (0, i, 0)),
        out_shape=jax.ShapeDtypeStruct((NS, N, D_OUT), jnp.float32),
    )(parts, W2, W2_comp)


def _softmax_body(p_ref, out_ref):
    s = p_ref[0] + p_ref[1]
    m = jnp.max(s, axis=1, keepdims=True)
    e = jnp.exp(s - m)
    out_ref[...] = e / jnp.sum(e, axis=1, keepdims=True)


def _softmax_sum(parts):
    return pl.pallas_call(
        _softmax_body,
        grid=(GRID,),
        in_specs=[pl.BlockSpec((NC, TN, D_OUT), lambda i: (0, i, 0))],
        out_specs=pl.BlockSpec((TN, D_OUT), lambda i: (i, 0)),
        out_shape=jax.ShapeDtypeStruct((N, D_OUT), jnp.float32),
    )(parts)


_MESH = plsc.VectorSubcoreMesh(core_axis_name="c", subcore_axis_name="s")


@functools.partial(
    pl.kernel,
    out_type=jax.ShapeDtypeStruct((4, NPAD, 64), jnp.float32),
    mesh=_MESH,
    compiler_params=pltpu.CompilerParams(use_tc_tiling_on_sc=False),
    scratch_types=[
        pltpu.VMEM((SLOTS1,), jnp.int32),
        pltpu.VMEM((SLOTS1,), jnp.int32),
        pltpu.VMEM((SLOTS1,), jnp.int32),
        pltpu.VMEM((2, NSUP1, LL1), jnp.int32),
        pltpu.VMEM((NSUP1, LL1), jnp.int32),
        pltpu.VMEM((LL1, 64), jnp.float32),
        pltpu.VMEM((ZR1, 64), jnp.float32),
        pltpu.VMEM_SHARED((NPAD + NT, 64), jnp.float32),
        pltpu.SemaphoreType.DMA,
    ],
)
def _agg1(y1_hbm, src_hbm, dst_hbm, type_hbm, out_hbm,
          srcv, dstv, typev, gidx, didx, rows, zbuf, acc, sem):
    c = lax.axis_index("c")
    s = lax.axis_index("s")
    ebase = s * EPT1

    pltpu.sync_copy(src_hbm.at[pl.ds(ebase, EPT1)], srcv.at[pl.ds(0, EPT1)])
    pltpu.sync_copy(dst_hbm.at[pl.ds(ebase, EPT1)], dstv.at[pl.ds(0, EPT1)])
    pltpu.sync_copy(type_hbm.at[pl.ds(ebase, EPT1)], typev.at[pl.ds(0, EPT1)])

    lane = lax.broadcasted_iota(jnp.int32, (LANES,), 0)
    zv = jnp.zeros((LANES,), jnp.float32)

    def fill(j, carry):
        for k in range(LL1 // LANES):
            off = j * LL1 + k * LANES
            sv = srcv[pl.ds(off, LANES)]
            tv = typev[pl.ds(off, LANES)]
            dv = dstv[pl.ds(off, LANES)]
            valid = (off + lane) < EPT1
            base = (tv * N + sv) * 4 + 2 * c
            gidx[0, j, pl.ds(k * LANES, LANES)] = jnp.where(valid, base, 0)
            gidx[1, j, pl.ds(k * LANES, LANES)] = jnp.where(valid, base + 1, 0)
            didx[j, pl.ds(k * LANES, LANES)] = jnp.where(valid, dv, NPAD + s)
        return carry

    lax.fori_loop(0, NSUP1, fill, 0)

    def zfill(i, carry):
        for k in range(64 // LANES):
            zbuf[i, pl.ds(k * LANES, LANES)] = zv
        return carry

    r0 = s * RPT
    for p in range(2):
        q = 2 * c + p

        lax.fori_loop(0, ZR1, zfill, 0)
        for z in range(RPT // ZR1):
            pltpu.sync_copy(zbuf, acc.at[pl.ds(r0 + z * ZR1, ZR1)])

        plsc.subcore_barrier()

        def step(m, carry):
            pltpu.async_copy(y1_hbm.at[gidx.at[p, m]], rows, sem).wait()
            pltpu.sync_copy(rows, acc.at[didx.at[m]], add=True)
            return carry

        lax.fori_loop(0, NSUP1, step, 0)

        plsc.subcore_barrier()

        for z in range(RPT // ZR1):
            pltpu.sync_copy(acc.at[pl.ds(r0 + z * ZR1, ZR1)], zbuf)
            pltpu.sync_copy(zbuf, out_hbm.at[q, pl.ds(r0 + z * ZR1, ZR1)])


@functools.partial(
    pl.kernel,
    out_type=jax.ShapeDtypeStruct((NC, NPAD, D_OUT), jnp.float32),
    mesh=_MESH,
    compiler_params=pltpu.CompilerParams(use_tc_tiling_on_sc=False),
    scratch_types=[
        pltpu.VMEM((SLOTS2,), jnp.int32),
        pltpu.VMEM((SLOTS2,), jnp.int32),
        pltpu.VMEM((SLOTS2,), jnp.int32),
        pltpu.VMEM((NSUP2, SB2 * K2), jnp.int32),
        pltpu.VMEM((NSUP2, SB2 * K2), jnp.int32),
        pltpu.VMEM((SB2 * K2, D_OUT), jnp.float32),
        pltpu.VMEM((RPT, D_OUT), jnp.float32),
        pltpu.VMEM_SHARED((NPAD, D_OUT), jnp.float32),
        pltpu.SemaphoreType.DMA,
    ],
)
def _agg2(y2_hbm, src_hbm, dst_hbm, type_hbm, out_hbm,
          srcv, dstv, typev, gidx, didx, rows, zbuf, acc, sem):
    c = lax.axis_index("c")
    s = lax.axis_index("s")
    ebase = c * EPC + s * EPT2

    pltpu.sync_copy(src_hbm.at[pl.ds(ebase, EPT2)], srcv.at[pl.ds(0, EPT2)])
    pltpu.sync_copy(dst_hbm.at[pl.ds(ebase, EPT2)], dstv.at[pl.ds(0, EPT2)])
    pltpu.sync_copy(type_hbm.at[pl.ds(ebase, EPT2)], typev.at[pl.ds(0, EPT2)])

    lane = lax.broadcasted_iota(jnp.int32, (LANES,), 0)
    zv = jnp.zeros((LANES,), jnp.float32)

    def fill(j, carry):
        jm = j // SB2
        jp = (j % SB2) * K2
        for k in range(K2 // LANES):
            off = j * K2 + k * LANES
            sv = srcv[pl.ds(off, LANES)]
            tv = typev[pl.ds(off, LANES)]
            dv = dstv[pl.ds(off, LANES)]
            valid = (off + lane) < EPT2
            gidx[jm, pl.ds(jp + k * LANES, LANES)] = jnp.where(valid, tv * N + sv, 0)
            didx[jm, pl.ds(jp + k * LANES, LANES)] = jnp.where(valid, dv, N + s)
        return carry

    lax.fori_loop(0, NCH2, fill, 0)

    def zfill(i, carry):
        zbuf[i] = zv
        return carry

    lax.fori_loop(0, RPT, zfill, 0)
    pltpu.sync_copy(zbuf, acc.at[pl.ds(s * RPT, RPT)])

    plsc.subcore_barrier()

    def step(m, carry):
        pltpu.async_copy(y2_hbm.at[gidx.at[m]], rows, sem).wait()
        pltpu.sync_copy(rows, acc.at[didx.at[m]], add=True)
        return carry

    lax.fori_loop(0, NSUP2, step, 0)

    plsc.subcore_barrier()

    pltpu.sync_copy(acc.at[pl.ds(s * RPT, RPT)], zbuf)
    pltpu.sync_copy(zbuf, out_hbm.at[c, pl.ds(s * RPT, RPT)])


def kernel(x, edge_index, edge_type, W1, W1_comp, W2, W2_comp):
    src = edge_index[0]
    dst = edge_index[1]
    y1 = _l1_matmul(x, W1, W1_comp)
    y1s = y1.reshape(NS * N * 4, 64)
    h1p = _agg1(y1s, src, dst, edge_type)
    y2 = _l2_matmul(h1p, W2, W2_comp)
    y2s = y2.reshape(NS * N, D_OUT)
    parts = _agg2(y2s, src, dst, edge_type)
    return _softmax_sum(parts)
